# baseline (speedup 1.0000x reference)
import functools

import jax
import jax.numpy as jnp
from jax import lax
from jax.experimental import pallas as pl
from jax.experimental.pallas import tpu as pltpu
from jax.experimental.pallas import tpu_sc as plsc

_N0 = 270336
_N1 = 24576
_N2 = 4096
_E0 = 245760
_E1 = 20480
_D = 128
_H = 128
_C = 41
_NC = 2
_NS = 16
_LN = 16
_HALF = _D // 2
_CH = 128


def _sc_aggregate(table2, init4, src2, dst2, n_out, n_edges, cb, nbuf):
    e_per_sub = n_edges // _NS
    n_blocks = e_per_sub // _CH
    nq = n_blocks // cb
    rps = n_out // _NS
    nz = rps // _CH
    mesh = plsc.VectorSubcoreMesh(core_axis_name="c", subcore_axis_name="s")

    @functools.partial(
        pl.kernel,
        out_type=jax.ShapeDtypeStruct((n_out, _D), jnp.float32),
        mesh=mesh,
        scratch_types=[
            pltpu.VMEM((2, cb, _CH), jnp.int32),
            pltpu.VMEM((2, cb, _CH), jnp.int32),
            pltpu.VMEM((nbuf, _CH, _HALF), jnp.float32),
            pltpu.VMEM_SHARED((n_out, _HALF), jnp.float32),
            pltpu.SemaphoreType.DMA((2,)),
            pltpu.SemaphoreType.DMA((nbuf,)),
            pltpu.SemaphoreType.DMA((nbuf,)),
        ],
        compiler_params=pltpu.CompilerParams(use_tc_tiling_on_sc=False),
    )
    def agg(table_hbm, init_hbm, src_hbm, dst_hbm, out_hbm, src_v, dst_v,
            rows_v, acc_sh, isem, gsem, ssem):
        c = lax.axis_index("c")
        s = lax.axis_index("s")

        def fire_idx(q, ib):
            pltpu.async_copy(
                src_hbm.at[c, pl.ds(s * n_blocks + q * cb, cb)],
                src_v.at[ib], isem.at[ib])
            pltpu.async_copy(
                dst_hbm.at[pl.ds(s * n_blocks + q * cb, cb)],
                dst_v.at[ib], isem.at[ib])

        def drain_idx(q, ib):
            pltpu.make_async_copy(
                src_hbm.at[c, pl.ds(s * n_blocks + q * cb, cb)],
                src_v.at[ib], isem.at[ib]).wait()
            pltpu.make_async_copy(
                dst_hbm.at[pl.ds(s * n_blocks + q * cb, cb)],
                dst_v.at[ib], isem.at[ib]).wait()

        fire_idx(0, 0)

        for t in range(nz):
            pltpu.async_copy(
                init_hbm.at[pl.ds((s * nz + t) * _CH, _CH), c],
                acc_sh.at[pl.ds((s * nz + t) * _CH, _CH)], ssem.at[0])

        drain_idx(0, 0)
        pre = [pltpu.async_copy(table_hbm.at[src_v.at[0, k]],
                                rows_v.at[k], gsem.at[k])
               for k in range(nbuf)]

        for t in range(nz):
            pltpu.make_async_copy(
                init_hbm.at[pl.ds(s * nz * _CH, _CH), c],
                acc_sh.at[pl.ds(s * nz * _CH, _CH)], ssem.at[0]).wait()

        plsc.subcore_barrier()

        for q in range(nq):
            ib = q % 2
            if q + 1 < nq:
                fire_idx(q + 1, 1 - ib)
            if q == 0:
                for k in range(nbuf):
                    pre[k].wait()
                    pltpu.async_copy(rows_v.at[k], acc_sh.at[dst_v.at[0, k]],
                                     ssem.at[k], add=True)
            else:
                drain_idx(q, ib)

            @pl.loop(1 if q == 0 else 0, cb // nbuf)
            def _(t):
                gathers = []
                for k in range(nbuf):
                    j = nbuf * t + k

                    @pl.when(t > 0)
                    def _():
                        pltpu.make_async_copy(
                            rows_v.at[k], acc_sh.at[dst_v.at[ib, j - nbuf]],
                            ssem.at[k]).wait()

                    gathers.append(pltpu.async_copy(
                        table_hbm.at[src_v.at[ib, j]],
                        rows_v.at[k], gsem.at[k]))
                for k in range(nbuf):
                    j = nbuf * t + k
                    gathers[k].wait()
                    pltpu.async_copy(rows_v.at[k], acc_sh.at[dst_v.at[ib, j]],
                                     ssem.at[k], add=True)

            for k in range(nbuf):
                pltpu.make_async_copy(
                    rows_v.at[k], acc_sh.at[dst_v.at[ib, cb - nbuf + k]],
                    ssem.at[k]).wait()

        plsc.subcore_barrier()

        pltpu.sync_copy(acc_sh.at[pl.ds(s * rps, rps)],
                        out_hbm.at[pl.ds(s * rps, rps), pl.ds(c * _HALF, _HALF)])

    return agg(table2, init4, src2, dst2)


def _edge_prep(ei0r, ei1r):
    def body(e0_ref, e1_ref, s0_ref, d0_ref, s1_ref, d1_ref):
        e0 = e0_ref[...]
        t0 = e0[0] * 2
        s0_ref[...] = jnp.stack([t0, t0 + 1])
        d0_ref[...] = e0[1]
        e1 = e1_ref[...]
        t1 = e1[0] * 2
        s1_ref[...] = jnp.stack([t1, t1 + 1])
        d1_ref[...] = e1[1]

    nb0, nb1 = _E0 // _CH, _E1 // _CH
    return pl.pallas_call(
        body,
        grid=(1,),
        in_specs=[
            pl.BlockSpec((2, nb0, _CH), lambda i: (0, 0, 0)),
            pl.BlockSpec((2, nb1, _CH), lambda i: (0, 0, 0)),
        ],
        out_specs=[
            pl.BlockSpec((2, nb0, _CH), lambda i: (0, 0, 0)),
            pl.BlockSpec((nb0, _CH), lambda i: (0, 0)),
            pl.BlockSpec((2, nb1, _CH), lambda i: (0, 0, 0)),
            pl.BlockSpec((nb1, _CH), lambda i: (0, 0)),
        ],
        out_shape=[
            jax.ShapeDtypeStruct((2, nb0, _CH), jnp.int32),
            jax.ShapeDtypeStruct((nb0, _CH), jnp.int32),
            jax.ShapeDtypeStruct((2, nb1, _CH), jnp.int32),
            jax.ShapeDtypeStruct((nb1, _CH), jnp.int32),
        ],
    )(ei0r, ei1r)


_BA = 2048


def _mlp0(aggr, W1, b1, g1, bt1, W2, b2):
    def body(a_ref, w1_ref, b1_ref, g_ref, bt_ref, w2_ref, b2_ref, o_ref):
        z = a_ref[...]
        h = jnp.dot(z, w1_ref[...], preferred_element_type=jnp.float32) + b1_ref[...]
        m = jnp.mean(h, axis=0, keepdims=True)
        v = jnp.mean((h - m) ** 2, axis=0, keepdims=True)
        h = (h - m) * (g_ref[...] / jnp.sqrt(v + 1e-5)) + bt_ref[...]
        h = jnp.maximum(h, 0.0)
        h = jnp.dot(h, w2_ref[...], preferred_element_type=jnp.float32) + b2_ref[...]
        o_ref[...] = jnp.maximum(h, 0.0)

    return pl.pallas_call(
        body,
        grid=(1,),
        in_specs=[
            pl.BlockSpec((_N1, _D), lambda i: (0, 0)),
            pl.BlockSpec((_D, _H), lambda i: (0, 0)),
            pl.BlockSpec((1, _H), lambda i: (0, 0)),
            pl.BlockSpec((1, _H), lambda i: (0, 0)),
            pl.BlockSpec((1, _H), lambda i: (0, 0)),
            pl.BlockSpec((_H, _H), lambda i: (0, 0)),
            pl.BlockSpec((1, _H), lambda i: (0, 0)),
        ],
        out_specs=pl.BlockSpec((_N1, _H), lambda i: (0, 0)),
        out_shape=jax.ShapeDtypeStruct((_N1, _H), jnp.float32),
    )(aggr, W1, b1, g1, bt1, W2, b2)


def _mlp1(aggr, W3, b3, g2, bt2, W4, b4):
    def body(a_ref, w3_ref, b3_ref, g_ref, bt_ref, w4_ref, b4_ref, o_ref):
        z = a_ref[...]
        t = jnp.dot(z, w3_ref[...], preferred_element_type=jnp.float32) + b3_ref[...]
        m = jnp.mean(t, axis=0, keepdims=True)
        v = jnp.mean((t - m) ** 2, axis=0, keepdims=True)
        t = (t - m) / jnp.sqrt(v + 1e-5) * g_ref[...] + bt_ref[...]
        t = jnp.maximum(t, 0.0)
        t = jnp.dot(t, w4_ref[...], preferred_element_type=jnp.float32) + b4_ref[...]
        t = t - jnp.max(t, axis=-1, keepdims=True)
        t = t - jnp.log(jnp.sum(jnp.exp(t), axis=-1, keepdims=True))
        o_ref[...] = t.T

    return pl.pallas_call(
        body,
        grid=(1,),
        in_specs=[
            pl.BlockSpec((_N2, _H), lambda i: (0, 0)),
            pl.BlockSpec((_H, _H), lambda i: (0, 0)),
            pl.BlockSpec((1, _H), lambda i: (0, 0)),
            pl.BlockSpec((1, _H), lambda i: (0, 0)),
            pl.BlockSpec((1, _H), lambda i: (0, 0)),
            pl.BlockSpec((_H, _C), lambda i: (0, 0)),
            pl.BlockSpec((1, _C), lambda i: (0, 0)),
        ],
        out_specs=pl.BlockSpec((_C, _N2), lambda i: (0, 0)),
        out_shape=jax.ShapeDtypeStruct((_C, _N2), jnp.float32),
    )(aggr, W3, b3, g2, bt2, W4, b4).T


def kernel(x_batch, edge_index0, edge_index1, size0, size1,
           W1, b1, g1, bt1, W2, b2, W3, b3, g2, bt2, W4, b4):
    x2 = x_batch.reshape(2 * _N0, _HALF)
    x4 = x_batch.reshape(_N0, 2, _HALF)
    src0, dst0, src1, dst1 = _edge_prep(
        edge_index0.reshape(2, _E0 // _CH, _CH),
        edge_index1.reshape(2, _E1 // _CH, _CH))
    z0 = _sc_aggregate(x2, x4, src0, dst0, _N1, _E0, 15, 3)
    h = _mlp0(z0, W1, b1.reshape(1, _H), g1.reshape(1, _H),
              bt1.reshape(1, _H), W2, b2.reshape(1, _H))
    h2 = h.reshape(2 * _N1, _HALF)
    h4 = h.reshape(_N1, 2, _HALF)
    z1 = _sc_aggregate(h2, h4, src1, dst1, _N2, _E1, 10, 5)
    return _mlp1(z1, W3, b3.reshape(1, _H), g2.reshape(1, _H),
                 bt2.reshape(1, _H), W4, b4.reshape(1, _C))


# speedup vs baseline: 6.1059x; 6.1059x over previous
import functools

import jax
import jax.numpy as jnp
from jax import lax
from jax.experimental import pallas as pl
from jax.experimental.pallas import tpu as pltpu
from jax.experimental.pallas import tpu_sc as plsc

_N0 = 270336
_N1 = 24576
_N2 = 4096
_E0 = 245760
_E1 = 20480
_D = 128
_H = 128
_C = 41
_NC = 2
_NS = 16
_LN = 16
_HALF = _D // 2
_CH = 128


def _sc_aggregate(table2, src2, dst2, n_out, n_edges, cb, nbuf):
    e_per_sub = n_edges // _NS
    n_blocks = e_per_sub // _CH
    nq = n_blocks // cb
    rps = n_out // _NS
    nz = rps // _CH
    mesh = plsc.VectorSubcoreMesh(core_axis_name="c", subcore_axis_name="s")

    @functools.partial(
        pl.kernel,
        out_type=jax.ShapeDtypeStruct((n_out, _D), jnp.float32),
        mesh=mesh,
        scratch_types=[
            pltpu.VMEM((2, cb, _CH), jnp.int32),
            pltpu.VMEM((2, cb, _CH), jnp.int32),
            pltpu.VMEM((nbuf, _CH, _HALF), jnp.float32),
            pltpu.VMEM_SHARED((n_out, _HALF), jnp.float32),
            pltpu.SemaphoreType.DMA((2,)),
            pltpu.SemaphoreType.DMA((nbuf,)),
            pltpu.SemaphoreType.DMA((nbuf,)),
        ],
        compiler_params=pltpu.CompilerParams(use_tc_tiling_on_sc=False),
    )
    def agg(table_hbm, src_hbm, dst_hbm, out_hbm, src_v, dst_v, rows_v,
            acc_sh, isem, gsem, ssem):
        c = lax.axis_index("c")
        s = lax.axis_index("s")

        def fire_idx(q, ib):
            pltpu.async_copy(
                src_hbm.at[c, pl.ds(s * n_blocks + q * cb, cb)],
                src_v.at[ib], isem.at[ib])
            pltpu.async_copy(
                dst_hbm.at[pl.ds(s * n_blocks + q * cb, cb)],
                dst_v.at[ib], isem.at[ib])

        def drain_idx(q, ib):
            pltpu.make_async_copy(
                src_hbm.at[c, pl.ds(s * n_blocks + q * cb, cb)],
                src_v.at[ib], isem.at[ib]).wait()
            pltpu.make_async_copy(
                dst_hbm.at[pl.ds(s * n_blocks + q * cb, cb)],
                dst_v.at[ib], isem.at[ib]).wait()

        fire_idx(0, 0)

        @pl.loop(0, _CH)
        def _(i):
            @pl.loop(0, _HALF, step=_LN)
            def _(j):
                rows_v[0, i, pl.ds(j, _LN)] = jnp.zeros((_LN,), jnp.float32)

        for t in range(nz):
            pltpu.async_copy(
                rows_v.at[0], acc_sh.at[pl.ds((s * nz + t) * _CH, _CH)],
                ssem.at[0])
        for t in range(nz):
            pltpu.make_async_copy(
                rows_v.at[0], acc_sh.at[pl.ds(s * nz * _CH, _CH)],
                ssem.at[0]).wait()

        plsc.subcore_barrier()

        for q in range(nq):
            ib = q % 2
            if q + 1 < nq:
                fire_idx(q + 1, 1 - ib)
            drain_idx(q, ib)

            @pl.loop(0, cb // nbuf)
            def _(t):
                gathers = []
                for k in range(nbuf):
                    j = nbuf * t + k

                    @pl.when(t > 0)
                    def _():
                        pltpu.make_async_copy(
                            rows_v.at[k], acc_sh.at[dst_v.at[ib, j - nbuf]],
                            ssem.at[k]).wait()

                    gathers.append(pltpu.async_copy(
                        table_hbm.at[src_v.at[ib, j]],
                        rows_v.at[k], gsem.at[k]))
                for k in range(nbuf):
                    j = nbuf * t + k
                    gathers[k].wait()
                    pltpu.async_copy(rows_v.at[k], acc_sh.at[dst_v.at[ib, j]],
                                     ssem.at[k], add=True)

            for k in range(nbuf):
                pltpu.make_async_copy(
                    rows_v.at[k], acc_sh.at[dst_v.at[ib, cb - nbuf + k]],
                    ssem.at[k]).wait()

        plsc.subcore_barrier()

        pltpu.sync_copy(acc_sh.at[pl.ds(s * rps, rps)],
                        out_hbm.at[pl.ds(s * rps, rps), pl.ds(c * _HALF, _HALF)])

    return agg(table2, src2, dst2)


def _edge_prep(ei0r, ei1r):
    def body(e0_ref, e1_ref, s0_ref, d0_ref, s1_ref, d1_ref):
        e0 = e0_ref[...]
        t0 = e0[0] * 2
        s0_ref[...] = jnp.stack([t0, t0 + 1])
        d0_ref[...] = e0[1]
        e1 = e1_ref[...]
        t1 = e1[0] * 2
        s1_ref[...] = jnp.stack([t1, t1 + 1])
        d1_ref[...] = e1[1]

    nb0, nb1 = _E0 // _CH, _E1 // _CH
    return pl.pallas_call(
        body,
        grid=(1,),
        in_specs=[
            pl.BlockSpec((2, nb0, _CH), lambda i: (0, 0, 0)),
            pl.BlockSpec((2, nb1, _CH), lambda i: (0, 0, 0)),
        ],
        out_specs=[
            pl.BlockSpec((2, nb0, _CH), lambda i: (0, 0, 0)),
            pl.BlockSpec((nb0, _CH), lambda i: (0, 0)),
            pl.BlockSpec((2, nb1, _CH), lambda i: (0, 0, 0)),
            pl.BlockSpec((nb1, _CH), lambda i: (0, 0)),
        ],
        out_shape=[
            jax.ShapeDtypeStruct((2, nb0, _CH), jnp.int32),
            jax.ShapeDtypeStruct((nb0, _CH), jnp.int32),
            jax.ShapeDtypeStruct((2, nb1, _CH), jnp.int32),
            jax.ShapeDtypeStruct((nb1, _CH), jnp.int32),
        ],
    )(ei0r, ei1r)


_BA = 2048


def _mlp0(x_batch, aggr, W1, b1, g1, bt1, W2, b2):
    def body(x_ref, a_ref, w1_ref, b1_ref, g_ref, bt_ref, w2_ref, b2_ref, o_ref):
        z = x_ref[...] + a_ref[...]
        h = jnp.dot(z, w1_ref[...], preferred_element_type=jnp.float32) + b1_ref[...]
        m = jnp.mean(h, axis=0, keepdims=True)
        v = jnp.mean((h - m) ** 2, axis=0, keepdims=True)
        h = (h - m) * (g_ref[...] / jnp.sqrt(v + 1e-5)) + bt_ref[...]
        h = jnp.maximum(h, 0.0)
        h = jnp.dot(h, w2_ref[...], preferred_element_type=jnp.float32) + b2_ref[...]
        o_ref[...] = jnp.maximum(h, 0.0)

    return pl.pallas_call(
        body,
        grid=(1,),
        in_specs=[
            pl.BlockSpec((_N1, _D), lambda i: (0, 0)),
            pl.BlockSpec((_N1, _D), lambda i: (0, 0)),
            pl.BlockSpec((_D, _H), lambda i: (0, 0)),
            pl.BlockSpec((1, _H), lambda i: (0, 0)),
            pl.BlockSpec((1, _H), lambda i: (0, 0)),
            pl.BlockSpec((1, _H), lambda i: (0, 0)),
            pl.BlockSpec((_H, _H), lambda i: (0, 0)),
            pl.BlockSpec((1, _H), lambda i: (0, 0)),
        ],
        out_specs=pl.BlockSpec((_N1, _H), lambda i: (0, 0)),
        out_shape=jax.ShapeDtypeStruct((_N1, _H), jnp.float32),
    )(x_batch, aggr, W1, b1, g1, bt1, W2, b2)


def _mlp1(h, aggr, W3, b3, g2, bt2, W4, b4):
    def body(h_ref, a_ref, w3_ref, b3_ref, g_ref, bt_ref, w4_ref, b4_ref, o_ref):
        z = h_ref[...] + a_ref[...]
        t = jnp.dot(z, w3_ref[...], preferred_element_type=jnp.float32) + b3_ref[...]
        m = jnp.mean(t, axis=0, keepdims=True)
        v = jnp.mean((t - m) ** 2, axis=0, keepdims=True)
        t = (t - m) / jnp.sqrt(v + 1e-5) * g_ref[...] + bt_ref[...]
        t = jnp.maximum(t, 0.0)
        t = jnp.dot(t, w4_ref[...], preferred_element_type=jnp.float32) + b4_ref[...]
        t = t - jnp.max(t, axis=-1, keepdims=True)
        t = t - jnp.log(jnp.sum(jnp.exp(t), axis=-1, keepdims=True))
        o_ref[...] = t.T

    return pl.pallas_call(
        body,
        grid=(1,),
        in_specs=[
            pl.BlockSpec((_N2, _H), lambda i: (0, 0)),
            pl.BlockSpec((_N2, _H), lambda i: (0, 0)),
            pl.BlockSpec((_H, _H), lambda i: (0, 0)),
            pl.BlockSpec((1, _H), lambda i: (0, 0)),
            pl.BlockSpec((1, _H), lambda i: (0, 0)),
            pl.BlockSpec((1, _H), lambda i: (0, 0)),
            pl.BlockSpec((_H, _C), lambda i: (0, 0)),
            pl.BlockSpec((1, _C), lambda i: (0, 0)),
        ],
        out_specs=pl.BlockSpec((_C, _N2), lambda i: (0, 0)),
        out_shape=jax.ShapeDtypeStruct((_C, _N2), jnp.float32),
    )(h, aggr, W3, b3, g2, bt2, W4, b4).T


def kernel(x_batch, edge_index0, edge_index1, size0, size1,
           W1, b1, g1, bt1, W2, b2, W3, b3, g2, bt2, W4, b4):
    x2 = x_batch.reshape(2 * _N0, _HALF)
    src0, dst0, src1, dst1 = _edge_prep(
        edge_index0.reshape(2, _E0 // _CH, _CH),
        edge_index1.reshape(2, _E1 // _CH, _CH))
    aggr0 = _sc_aggregate(x2, src0, dst0, _N1, _E0, 15, 3)
    h = _mlp0(x_batch, aggr0, W1, b1.reshape(1, _H), g1.reshape(1, _H),
              bt1.reshape(1, _H), W2, b2.reshape(1, _H))
    h2 = h.reshape(2 * _N1, _HALF)
    aggr1 = _sc_aggregate(h2, src1, dst1, _N2, _E1, 10, 5)
    return _mlp1(h, aggr1, W3, b3.reshape(1, _H), g2.reshape(1, _H),
                 bt2.reshape(1, _H), W4, b4.reshape(1, _C))
